# baseline (speedup 1.0000x reference)
import functools
import jax
import jax.numpy as jnp
from jax import lax
from jax.experimental import pallas as pl
from jax.experimental.pallas import tpu as pltpu
from jax.experimental.pallas import tpu_sc as plsc

_M = 0.2
_LYMBDA = 1.0
_NEG = float("-inf")
_L = 16
_NW = 32


def _sig(x):
    return 1.0 / (1.0 + jnp.exp(-x))


def _sc_body(inters_h, rels_h, labels_h, mem_h, rl_h, gt_h, mw_h,
             m1_h, xl_h, m2_h, xr_h,
             xbuf, rbuf, mw_v, bias_v, lab_v, gt_v, mm_v, rlv_v,
             o_m1, o_xl, o_m2, o_xr, sem_x0, sem_x1,
             *, t, c, nr, nb):
    cid = lax.axis_index("c")
    sid = lax.axis_index("s")
    wid = sid * 2 + cid
    b0 = wid * nb

    pltpu.sync_copy(labels_h.at[pl.ds(b0, nb)], lab_v.at[pl.ds(0, nb)])
    pltpu.sync_copy(gt_h.at[pl.ds(b0, nb), :], gt_v.at[:, pl.ds(0, 2)])
    pltpu.sync_copy(mem_h.at[pl.ds(b0, nb), :], mm_v)
    pltpu.sync_copy(rl_h.at[pl.ds(b0, nb), :], rlv_v.at[:, pl.ds(0, t)])

    tio = lax.iota(jnp.int32, _L)
    ninf16 = jnp.full((_L,), _NEG, jnp.float32)
    ngroups = t // _L

    pltpu.async_copy(inters_h.at[b0, pl.ds(0, _L), :], xbuf.at[0], sem_x0)

    def b_body(i, carry):
        b = b0 + i
        pltpu.sync_copy(mw_h.at[b, :], mw_v)
        pltpu.sync_copy(rels_h.at[b], rbuf)
        lab_s = lab_v[pl.ds(i, _L)][0]
        gtrow = gt_v[i, pl.ds(0, _L)]
        g0s = gtrow[0]
        g1s = gtrow[1]
        t0 = rlv_v[i, pl.ds(g0s, _L)][0]
        t1 = rlv_v[i, pl.ds(g1s, _L)][0]
        t0c = jnp.minimum(t0, nr - 1)

        def bias_body(j, carry2):
            mwch = mw_v[pl.ds(j * _L, _L)]
            ci = lax.iota(jnp.int32, _L) + j * _L
            bias_v[pl.ds(j * _L, _L)] = jnp.where(
                (mwch > 0) & (ci != lab_s), 0.0, _NEG)
            return carry2

        lax.fori_loop(0, c // _L, bias_body, 0)

        for g in range(ngroups):
            buf = g % 2
            sem = sem_x0 if buf == 0 else sem_x1
            pltpu.make_async_copy(
                inters_h.at[b, pl.ds(g * _L, _L), :], xbuf.at[buf], sem
            ).wait()
            if g < ngroups - 1:
                nbuf = (g + 1) % 2
                nsem = sem_x0 if nbuf == 0 else sem_x1
                pltpu.async_copy(
                    inters_h.at[b, pl.ds((g + 1) * _L, _L), :],
                    xbuf.at[nbuf], nsem)
            else:
                @pl.when(i + 1 < nb)
                def _():
                    pltpu.async_copy(
                        inters_h.at[b + 1, pl.ds(0, _L), :], xbuf.at[0],
                        sem_x0)
            xb = xbuf.at[buf]
            mm_chunk = mm_v[i, pl.ds(g * _L, _L)]
            rl_chunk = rlv_v[i, pl.ds(g * _L, _L)]
            memok = mm_chunk > 0

            def cbody(j, acc):
                bch = bias_v[pl.ds(j * _L, _L)]
                for u in range(_L):
                    gx = plsc.load_gather(
                        xb, [tio, jnp.full((_L,), j * _L + u, jnp.int32)])
                    acc = jnp.maximum(acc, gx + bch[u])
                return acc

            acc = lax.fori_loop(0, c // _L, cbody, ninf16)
            m1v = jnp.where(memok, acc, _NEG)
            gxl = plsc.load_gather(xb, [tio, jnp.full((_L,), lab_s,
                                                      jnp.int32)])
            xlv = jnp.where(memok, gxl, _NEG)

            rtio = tio + g * _L

            def rbody(j, racc):
                for u in range(_L):
                    cc = j * _L + u
                    gr = plsc.load_gather(
                        rbuf, [rtio, jnp.full((_L,), cc, jnp.int32)])
                    bias = jnp.where((cc != t0) & (cc != t1), 0.0, _NEG)
                    racc = jnp.maximum(racc, gr + bias)
                return racc

            racc = lax.fori_loop(0, nr // _L, rbody, ninf16)
            rowok = memok & (rl_chunk != nr)
            m2v = jnp.where(rowok, racc, _NEG)
            gxr = plsc.load_gather(rbuf, [rtio, jnp.full((_L,), t0c,
                                                         jnp.int32)])
            xrv = jnp.where(rowok & (t0 != nr), gxr, _NEG)

            o_m1[i, pl.ds(g * _L, _L)] = m1v
            o_xl[i, pl.ds(g * _L, _L)] = xlv
            o_m2[i, pl.ds(g * _L, _L)] = m2v
            o_xr[i, pl.ds(g * _L, _L)] = xrv
        return carry

    lax.fori_loop(0, nb, b_body, 0)

    pltpu.sync_copy(o_m1, m1_h.at[pl.ds(b0, nb), :])
    pltpu.sync_copy(o_xl, xl_h.at[pl.ds(b0, nb), :])
    pltpu.sync_copy(o_m2, m2_h.at[pl.ds(b0, nb), :])
    pltpu.sync_copy(o_xr, xr_h.at[pl.ds(b0, nb), :])


def _sc_reduce(inters, rels, labels, mem_mask, rels_label, gt_tracks,
               multilab_weights):
    b, t, c = inters.shape
    nr = rels.shape[2]
    nb = b // _NW
    mesh = plsc.VectorSubcoreMesh(core_axis_name="c", subcore_axis_name="s")
    body = functools.partial(_sc_body, t=t, c=c, nr=nr, nb=nb)
    f = pl.kernel(
        body,
        out_type=[jax.ShapeDtypeStruct((b, t), jnp.float32)] * 4,
        mesh=mesh,
        compiler_params=pltpu.CompilerParams(
            use_tc_tiling_on_sc=False, needs_layout_passes=False),
        scratch_types=[
            pltpu.VMEM((2, _L, c), jnp.float32),
            pltpu.VMEM((t, nr), jnp.float32),
            pltpu.VMEM((c,), jnp.int32),
            pltpu.VMEM((c,), jnp.float32),
            pltpu.VMEM((nb + _L,), jnp.int32),
            pltpu.VMEM((nb, _L), jnp.int32),
            pltpu.VMEM((nb, t), jnp.int32),
            pltpu.VMEM((nb, t + _L), jnp.int32),
            pltpu.VMEM((nb, t), jnp.float32),
            pltpu.VMEM((nb, t), jnp.float32),
            pltpu.VMEM((nb, t), jnp.float32),
            pltpu.VMEM((nb, t), jnp.float32),
            pltpu.SemaphoreType.DMA,
            pltpu.SemaphoreType.DMA,
        ],
    )
    return f(inters, rels, labels, mem_mask, rels_label, gt_tracks,
             multilab_weights)


def _tail_body(m1_ref, xl_ref, m2_ref, xr_ref, mem_ref, out_ref, *, b, t,
               inv_b):
    memb = mem_ref[...] > 0
    s_xl = _sig(xl_ref[...])
    s_xr = _sig(xr_ref[...])
    mv = (s_xl + s_xr) * memb.astype(jnp.float32)
    titer = lax.broadcasted_iota(jnp.int32, (b, t), 1)
    maxv = jnp.max(mv, axis=1)
    ismax = mv == maxv[:, None]
    first = jnp.min(jnp.where(ismax, titer, t), axis=1)
    sel = titer == first[:, None]
    pos = jnp.max(jnp.where(sel, s_xl, 0.0), axis=1)
    pos_r = jnp.max(jnp.where(sel, s_xr, 0.0), axis=1)
    term = (_LYMBDA * jnp.maximum(_M - pos[:, None] + _sig(m1_ref[...]), 0.0)
            + jnp.maximum(_M - pos_r[:, None] + _sig(m2_ref[...]), 0.0))
    out_ref[...] = jnp.full((1, 1), jnp.sum(term) * inv_b, jnp.float32)


def _tail(m1, xl, m2, xr, mem_mask):
    b, t = m1.shape
    body = functools.partial(_tail_body, b=b, t=t, inv_b=1.0 / b)
    return pl.pallas_call(
        body,
        out_shape=jax.ShapeDtypeStruct((1, 1), jnp.float32),
    )(m1, xl, m2, xr, mem_mask)


@jax.jit
def kernel(inters, rels, labels, mem_mask, rels_label, gt_tracks,
           multilab_weights):
    m1, xl, m2, xr = _sc_reduce(inters, rels, labels, mem_mask, rels_label,
                                gt_tracks, multilab_weights)
    out = _tail(m1, xl, m2, xr, mem_mask)
    return out.reshape((1,))


# speedup vs baseline: 3.0404x; 3.0404x over previous
import functools
import jax
import jax.numpy as jnp
from jax import lax
from jax.experimental import pallas as pl
from jax.experimental.pallas import tpu as pltpu
from jax.experimental.pallas import tpu_sc as plsc

_M = 0.2
_LYMBDA = 1.0
_NEG = float("-inf")
_L = 16
_NW = 32


def _sig(x):
    return 1.0 / (1.0 + jnp.exp(-x))


def _sc_body(inters_h, rels_h, labels_h, mem_h, rl_h, gt_h, mw_h,
             m1_h, xl_h, m2_h, xr_h,
             xbuf, rbuf, mw_v, bias_v, rbias_v, lab_v, gt_v, mm_v, rlv_v,
             o_m1, o_xl, o_m2, o_xr, sem_x0, sem_x1,
             *, t, c, nr, nb):
    cid = lax.axis_index("c")
    sid = lax.axis_index("s")
    wid = sid * 2 + cid
    b0 = wid * nb

    pltpu.sync_copy(labels_h.at[pl.ds(b0, nb)], lab_v.at[pl.ds(0, nb)])
    pltpu.sync_copy(gt_h.at[pl.ds(b0, nb), :], gt_v.at[:, pl.ds(0, 2)])
    pltpu.sync_copy(mem_h.at[pl.ds(b0, nb), :], mm_v)
    pltpu.sync_copy(rl_h.at[pl.ds(b0, nb), :], rlv_v.at[:, pl.ds(0, t)])

    tio = lax.iota(jnp.int32, _L)
    ninf16 = jnp.full((_L,), _NEG, jnp.float32)
    ngroups = t // _L

    pltpu.async_copy(inters_h.at[b0, pl.ds(0, _L), :], xbuf.at[0], sem_x0)

    def b_body(i, carry):
        b = b0 + i
        pltpu.sync_copy(mw_h.at[b, :], mw_v)
        pltpu.sync_copy(rels_h.at[b], rbuf)
        lab_s = lab_v[pl.ds(i, _L)][0]
        gtrow = gt_v[i, pl.ds(0, _L)]
        g0s = gtrow[0]
        g1s = gtrow[1]
        t0 = rlv_v[i, pl.ds(g0s, _L)][0]
        t1 = rlv_v[i, pl.ds(g1s, _L)][0]
        t0c = jnp.minimum(t0, nr - 1)

        def bias_body(j, carry2):
            mwch = mw_v[pl.ds(j * _L, _L)]
            ci = lax.iota(jnp.int32, _L) + j * _L
            bias_v[pl.ds(j * _L, _L)] = jnp.where(
                (mwch > 0) & (ci != lab_s), 0.0, _NEG)
            return carry2

        lax.fori_loop(0, c // _L, bias_body, 0)

        for j in range(nr // _L):
            ci = lax.iota(jnp.int32, _L) + j * _L
            rbias_v[pl.ds(j * _L, _L)] = jnp.where(
                (ci != t0) & (ci != t1), 0.0, _NEG)

        for g in range(ngroups):
            buf = g % 2
            sem = sem_x0 if buf == 0 else sem_x1
            pltpu.make_async_copy(
                inters_h.at[b, pl.ds(g * _L, _L), :], xbuf.at[buf], sem
            ).wait()
            if g < ngroups - 1:
                nbuf = (g + 1) % 2
                nsem = sem_x0 if nbuf == 0 else sem_x1
                pltpu.async_copy(
                    inters_h.at[b, pl.ds((g + 1) * _L, _L), :],
                    xbuf.at[nbuf], nsem)
            else:
                @pl.when(i + 1 < nb)
                def _():
                    pltpu.async_copy(
                        inters_h.at[b + 1, pl.ds(0, _L), :], xbuf.at[0],
                        sem_x0)
            xb = xbuf.at[buf]
            mm_chunk = mm_v[i, pl.ds(g * _L, _L)]
            rl_chunk = rlv_v[i, pl.ds(g * _L, _L)]
            memok = mm_chunk > 0

            def cbody(j, accs):
                bch = bias_v[pl.ds(j * _L, _L)]
                out = []
                for tt in range(_L):
                    xv = xb[tt, pl.ds(j * _L, _L)]
                    out.append(jnp.maximum(accs[tt], xv + bch))
                return tuple(out)

            accs = lax.fori_loop(0, c // _L, cbody, (ninf16,) * _L)
            m1v = ninf16
            for tt in range(_L):
                m1v = jnp.where(tio == tt, jnp.max(accs[tt]), m1v)
            m1v = jnp.where(memok, m1v, _NEG)
            gxl = plsc.load_gather(xb, [tio, jnp.full((_L,), lab_s,
                                                      jnp.int32)])
            xlv = jnp.where(memok, gxl, _NEG)

            rtio = tio + g * _L

            raccs = [ninf16] * _L
            for j in range(nr // _L):
                rbch = rbias_v[pl.ds(j * _L, _L)]
                for tt in range(_L):
                    rv = rbuf[g * _L + tt, pl.ds(j * _L, _L)]
                    raccs[tt] = jnp.maximum(raccs[tt], rv + rbch)
            m2v = ninf16
            for tt in range(_L):
                m2v = jnp.where(tio == tt, jnp.max(raccs[tt]), m2v)
            rowok = memok & (rl_chunk != nr)
            m2v = jnp.where(rowok, m2v, _NEG)
            gxr = plsc.load_gather(rbuf, [rtio, jnp.full((_L,), t0c,
                                                         jnp.int32)])
            xrv = jnp.where(rowok & (t0 != nr), gxr, _NEG)

            o_m1[i, pl.ds(g * _L, _L)] = m1v
            o_xl[i, pl.ds(g * _L, _L)] = xlv
            o_m2[i, pl.ds(g * _L, _L)] = m2v
            o_xr[i, pl.ds(g * _L, _L)] = xrv
        return carry

    lax.fori_loop(0, nb, b_body, 0)

    pltpu.sync_copy(o_m1, m1_h.at[pl.ds(b0, nb), :])
    pltpu.sync_copy(o_xl, xl_h.at[pl.ds(b0, nb), :])
    pltpu.sync_copy(o_m2, m2_h.at[pl.ds(b0, nb), :])
    pltpu.sync_copy(o_xr, xr_h.at[pl.ds(b0, nb), :])


def _sc_reduce(inters, rels, labels, mem_mask, rels_label, gt_tracks,
               multilab_weights):
    b, t, c = inters.shape
    nr = rels.shape[2]
    nb = b // _NW
    mesh = plsc.VectorSubcoreMesh(core_axis_name="c", subcore_axis_name="s")
    body = functools.partial(_sc_body, t=t, c=c, nr=nr, nb=nb)
    f = pl.kernel(
        body,
        out_type=[jax.ShapeDtypeStruct((b, t), jnp.float32)] * 4,
        mesh=mesh,
        compiler_params=pltpu.CompilerParams(
            use_tc_tiling_on_sc=False, needs_layout_passes=False),
        scratch_types=[
            pltpu.VMEM((2, _L, c), jnp.float32),
            pltpu.VMEM((t, nr), jnp.float32),
            pltpu.VMEM((c,), jnp.int32),
            pltpu.VMEM((c,), jnp.float32),
            pltpu.VMEM((nr,), jnp.float32),
            pltpu.VMEM((nb + _L,), jnp.int32),
            pltpu.VMEM((nb, _L), jnp.int32),
            pltpu.VMEM((nb, t), jnp.int32),
            pltpu.VMEM((nb, t + _L), jnp.int32),
            pltpu.VMEM((nb, t), jnp.float32),
            pltpu.VMEM((nb, t), jnp.float32),
            pltpu.VMEM((nb, t), jnp.float32),
            pltpu.VMEM((nb, t), jnp.float32),
            pltpu.SemaphoreType.DMA,
            pltpu.SemaphoreType.DMA,
        ],
    )
    return f(inters, rels, labels, mem_mask, rels_label, gt_tracks,
             multilab_weights)


def _tail_body(m1_ref, xl_ref, m2_ref, xr_ref, mem_ref, out_ref, *, b, t,
               inv_b):
    memb = mem_ref[...] > 0
    s_xl = _sig(xl_ref[...])
    s_xr = _sig(xr_ref[...])
    mv = (s_xl + s_xr) * memb.astype(jnp.float32)
    titer = lax.broadcasted_iota(jnp.int32, (b, t), 1)
    maxv = jnp.max(mv, axis=1)
    ismax = mv == maxv[:, None]
    first = jnp.min(jnp.where(ismax, titer, t), axis=1)
    sel = titer == first[:, None]
    pos = jnp.max(jnp.where(sel, s_xl, 0.0), axis=1)
    pos_r = jnp.max(jnp.where(sel, s_xr, 0.0), axis=1)
    term = (_LYMBDA * jnp.maximum(_M - pos[:, None] + _sig(m1_ref[...]), 0.0)
            + jnp.maximum(_M - pos_r[:, None] + _sig(m2_ref[...]), 0.0))
    out_ref[...] = jnp.full((1, 1), jnp.sum(term) * inv_b, jnp.float32)


def _tail(m1, xl, m2, xr, mem_mask):
    b, t = m1.shape
    body = functools.partial(_tail_body, b=b, t=t, inv_b=1.0 / b)
    return pl.pallas_call(
        body,
        out_shape=jax.ShapeDtypeStruct((1, 1), jnp.float32),
    )(m1, xl, m2, xr, mem_mask)


@jax.jit
def kernel(inters, rels, labels, mem_mask, rels_label, gt_tracks,
           multilab_weights):
    m1, xl, m2, xr = _sc_reduce(inters, rels, labels, mem_mask, rels_label,
                                gt_tracks, multilab_weights)
    out = _tail(m1, xl, m2, xr, mem_mask)
    return out.reshape((1,))


# speedup vs baseline: 3.4368x; 1.1304x over previous
import functools
import jax
import jax.numpy as jnp
from jax import lax
from jax.experimental import pallas as pl
from jax.experimental.pallas import tpu as pltpu
from jax.experimental.pallas import tpu_sc as plsc

_M = 0.2
_LYMBDA = 1.0
_NEG = float("-inf")
_L = 16
_NW = 32
_B_SC = 256


def _sig(x):
    return 1.0 / (1.0 + jnp.exp(-x))


def _sc_body(inters_h, rels_h, labels_h, mem_h, rl_h, gt_h, mw_h,
             m1_h, xl_h, m2_h, xr_h,
             xbuf, rbuf, mw_v, bias_v, rbias_v, lab_v, gt_v, mm_v, rlv_v,
             o_m1, o_xl, o_m2, o_xr, sem_x0, sem_x1,
             *, t, c, nr, nb):
    cid = lax.axis_index("c")
    sid = lax.axis_index("s")
    wid = sid * 2 + cid
    b0 = wid * nb

    pltpu.sync_copy(labels_h.at[pl.ds(b0, nb)], lab_v.at[pl.ds(0, nb)])
    pltpu.sync_copy(gt_h.at[pl.ds(b0, nb), :], gt_v.at[:, pl.ds(0, 2)])
    pltpu.sync_copy(mem_h.at[pl.ds(b0, nb), :], mm_v)
    pltpu.sync_copy(rl_h.at[pl.ds(b0, nb), :], rlv_v.at[:, pl.ds(0, t)])

    tio = lax.iota(jnp.int32, _L)
    ninf16 = jnp.full((_L,), _NEG, jnp.float32)
    ngroups = t // _L

    pltpu.async_copy(inters_h.at[b0, pl.ds(0, _L), :], xbuf.at[0], sem_x0)

    def b_body(i, carry):
        b = b0 + i
        pltpu.sync_copy(mw_h.at[b, :], mw_v)
        pltpu.sync_copy(rels_h.at[b], rbuf)
        lab_s = lab_v[pl.ds(i, _L)][0]
        gtrow = gt_v[i, pl.ds(0, _L)]
        g0s = gtrow[0]
        g1s = gtrow[1]
        t0 = rlv_v[i, pl.ds(g0s, _L)][0]
        t1 = rlv_v[i, pl.ds(g1s, _L)][0]
        t0c = jnp.minimum(t0, nr - 1)

        def bias_body(j, carry2):
            mwch = mw_v[pl.ds(j * _L, _L)]
            ci = lax.iota(jnp.int32, _L) + j * _L
            bias_v[pl.ds(j * _L, _L)] = jnp.where(
                (mwch > 0) & (ci != lab_s), 0.0, _NEG)
            return carry2

        lax.fori_loop(0, c // _L, bias_body, 0)

        for j in range(nr // _L):
            ci = lax.iota(jnp.int32, _L) + j * _L
            rbias_v[pl.ds(j * _L, _L)] = jnp.where(
                (ci != t0) & (ci != t1), 0.0, _NEG)

        for g in range(ngroups):
            buf = g % 2
            sem = sem_x0 if buf == 0 else sem_x1
            pltpu.make_async_copy(
                inters_h.at[b, pl.ds(g * _L, _L), :], xbuf.at[buf], sem
            ).wait()
            if g < ngroups - 1:
                nbuf = (g + 1) % 2
                nsem = sem_x0 if nbuf == 0 else sem_x1
                pltpu.async_copy(
                    inters_h.at[b, pl.ds((g + 1) * _L, _L), :],
                    xbuf.at[nbuf], nsem)
            else:
                @pl.when(i + 1 < nb)
                def _():
                    pltpu.async_copy(
                        inters_h.at[b + 1, pl.ds(0, _L), :], xbuf.at[0],
                        sem_x0)
            xb = xbuf.at[buf]
            mm_chunk = mm_v[i, pl.ds(g * _L, _L)]
            rl_chunk = rlv_v[i, pl.ds(g * _L, _L)]
            memok = mm_chunk > 0

            def cbody(j, accs):
                bch = bias_v[pl.ds(j * _L, _L)]
                out = []
                for tt in range(_L):
                    xv = xb[tt, pl.ds(j * _L, _L)]
                    out.append(jnp.maximum(accs[tt], xv + bch))
                return tuple(out)

            accs = lax.fori_loop(0, c // _L, cbody, (ninf16,) * _L)
            m1v = ninf16
            for tt in range(_L):
                m1v = jnp.where(tio == tt, jnp.max(accs[tt]), m1v)
            m1v = jnp.where(memok, m1v, _NEG)
            gxl = plsc.load_gather(xb, [tio, jnp.full((_L,), lab_s,
                                                      jnp.int32)])
            xlv = jnp.where(memok, gxl, _NEG)

            rtio = tio + g * _L

            raccs = [ninf16] * _L
            for j in range(nr // _L):
                rbch = rbias_v[pl.ds(j * _L, _L)]
                for tt in range(_L):
                    rv = rbuf[g * _L + tt, pl.ds(j * _L, _L)]
                    raccs[tt] = jnp.maximum(raccs[tt], rv + rbch)
            m2v = ninf16
            for tt in range(_L):
                m2v = jnp.where(tio == tt, jnp.max(raccs[tt]), m2v)
            rowok = memok & (rl_chunk != nr)
            m2v = jnp.where(rowok, m2v, _NEG)
            gxr = plsc.load_gather(rbuf, [rtio, jnp.full((_L,), t0c,
                                                         jnp.int32)])
            xrv = jnp.where(rowok & (t0 != nr), gxr, _NEG)

            o_m1[i, pl.ds(g * _L, _L)] = m1v
            o_xl[i, pl.ds(g * _L, _L)] = xlv
            o_m2[i, pl.ds(g * _L, _L)] = m2v
            o_xr[i, pl.ds(g * _L, _L)] = xrv
        return carry

    lax.fori_loop(0, nb, b_body, 0)

    pltpu.sync_copy(o_m1, m1_h.at[pl.ds(b0, nb), :])
    pltpu.sync_copy(o_xl, xl_h.at[pl.ds(b0, nb), :])
    pltpu.sync_copy(o_m2, m2_h.at[pl.ds(b0, nb), :])
    pltpu.sync_copy(o_xr, xr_h.at[pl.ds(b0, nb), :])


def _sc_reduce(inters, rels, labels, mem_mask, rels_label, gt_tracks,
               multilab_weights, b_sc):
    b, t, c = inters.shape
    nr = rels.shape[2]
    nb = b_sc // _NW
    mesh = plsc.VectorSubcoreMesh(core_axis_name="c", subcore_axis_name="s")
    body = functools.partial(_sc_body, t=t, c=c, nr=nr, nb=nb)
    f = pl.kernel(
        body,
        out_type=[jax.ShapeDtypeStruct((b_sc, t), jnp.float32)] * 4,
        mesh=mesh,
        compiler_params=pltpu.CompilerParams(
            use_tc_tiling_on_sc=False, needs_layout_passes=False),
        scratch_types=[
            pltpu.VMEM((2, _L, c), jnp.float32),
            pltpu.VMEM((t, nr), jnp.float32),
            pltpu.VMEM((c,), jnp.int32),
            pltpu.VMEM((c,), jnp.float32),
            pltpu.VMEM((nr,), jnp.float32),
            pltpu.VMEM((nb + _L,), jnp.int32),
            pltpu.VMEM((nb, _L), jnp.int32),
            pltpu.VMEM((nb, t), jnp.int32),
            pltpu.VMEM((nb, t + _L), jnp.int32),
            pltpu.VMEM((nb, t), jnp.float32),
            pltpu.VMEM((nb, t), jnp.float32),
            pltpu.VMEM((nb, t), jnp.float32),
            pltpu.VMEM((nb, t), jnp.float32),
            pltpu.SemaphoreType.DMA,
            pltpu.SemaphoreType.DMA,
        ],
    )
    return f(inters, rels, labels, mem_mask, rels_label, gt_tracks,
             multilab_weights)


def _tc_body(inters_ref, rels_ref, labels_ref, mem_ref, rl_ref, gt_ref,
             mw_ref, out_ref, *, bb, t, c, nr, inv_b):
    x = inters_ref[...]
    memi = mem_ref[...]
    memb = memi > 0
    mem3 = memi[:, :, None] > 0
    lab = labels_ref[...][:, 0]
    mw3 = mw_ref[...][:, None, :] > 0

    citer = lax.broadcasted_iota(jnp.int32, (bb, t, c), 2)
    tgt = citer == lab[:, None, None]
    negmask = mem3 & mw3 & (~tgt)
    m1 = jnp.max(jnp.where(negmask, x, _NEG), axis=2)
    xl = jnp.max(jnp.where(tgt & mem3, x, _NEG), axis=2)

    r = rels_ref[...]
    rl = rl_ref[...]
    rfi = memi * (rl != nr).astype(jnp.int32)
    rf3 = rfi[:, :, None] > 0
    g0 = gt_ref[...][:, 0]
    g1 = gt_ref[...][:, 1]
    titer = lax.broadcasted_iota(jnp.int32, (bb, t), 1)
    rel_t0 = jnp.sum(jnp.where(titer == g0[:, None], rl, 0), axis=1)
    rel_t1 = jnp.sum(jnp.where(titer == g1[:, None], rl, 0), axis=1)

    riter = lax.broadcasted_iota(jnp.int32, (bb, t, nr), 2)
    rneg = (rf3 & (riter != rel_t0[:, None, None])
            & (riter != rel_t1[:, None, None]))
    m2 = jnp.max(jnp.where(rneg, r, _NEG), axis=2)
    xr = jnp.max(jnp.where(rf3 & (riter == rel_t0[:, None, None]), r, _NEG),
                 axis=2)

    s_xl = _sig(xl)
    s_xr = _sig(xr)
    mv = (s_xl + s_xr) * memb.astype(jnp.float32)
    maxv = jnp.max(mv, axis=1)
    ismax = mv == maxv[:, None]
    first = jnp.min(jnp.where(ismax, titer, t), axis=1)
    sel = titer == first[:, None]
    pos = jnp.max(jnp.where(sel, s_xl, 0.0), axis=1)
    pos_r = jnp.max(jnp.where(sel, s_xr, 0.0), axis=1)

    term = (_LYMBDA * jnp.maximum(_M - pos[:, None] + _sig(m1), 0.0)
            + jnp.maximum(_M - pos_r[:, None] + _sig(m2), 0.0))
    partial = jnp.full((1, 1), jnp.sum(term) * inv_b, jnp.float32)

    @pl.when(pl.program_id(0) == 0)
    def _init():
        out_ref[...] = jnp.zeros((1, 1), jnp.float32)

    out_ref[...] += partial


def _tc_part(inters, rels, labels, mem_mask, rels_label, gt_tracks,
             multilab_weights, b_sc, inv_b):
    b, t, c = inters.shape
    nr = rels.shape[2]
    bb = 64
    k0 = b_sc // bb
    grid = ((b - b_sc) // bb,)
    body = functools.partial(_tc_body, bb=bb, t=t, c=c, nr=nr, inv_b=inv_b)
    return pl.pallas_call(
        body,
        grid=grid,
        in_specs=[
            pl.BlockSpec((bb, t, c), lambda i: (i + k0, 0, 0)),
            pl.BlockSpec((bb, t, nr), lambda i: (i + k0, 0, 0)),
            pl.BlockSpec((bb, 1), lambda i: (i + k0, 0)),
            pl.BlockSpec((bb, t), lambda i: (i + k0, 0)),
            pl.BlockSpec((bb, t), lambda i: (i + k0, 0)),
            pl.BlockSpec((bb, 2), lambda i: (i + k0, 0)),
            pl.BlockSpec((bb, c), lambda i: (i + k0, 0)),
        ],
        out_specs=pl.BlockSpec((1, 1), lambda i: (0, 0)),
        out_shape=jax.ShapeDtypeStruct((1, 1), jnp.float32),
    )(inters, rels, labels[:, None], mem_mask, rels_label, gt_tracks,
      multilab_weights)


def _tail_body(m1_ref, xl_ref, m2_ref, xr_ref, mem_ref, out_ref, *, b, t,
               inv_b):
    memb = mem_ref[...] > 0
    s_xl = _sig(xl_ref[...])
    s_xr = _sig(xr_ref[...])
    mv = (s_xl + s_xr) * memb.astype(jnp.float32)
    titer = lax.broadcasted_iota(jnp.int32, (b, t), 1)
    maxv = jnp.max(mv, axis=1)
    ismax = mv == maxv[:, None]
    first = jnp.min(jnp.where(ismax, titer, t), axis=1)
    sel = titer == first[:, None]
    pos = jnp.max(jnp.where(sel, s_xl, 0.0), axis=1)
    pos_r = jnp.max(jnp.where(sel, s_xr, 0.0), axis=1)
    term = (_LYMBDA * jnp.maximum(_M - pos[:, None] + _sig(m1_ref[...]), 0.0)
            + jnp.maximum(_M - pos_r[:, None] + _sig(m2_ref[...]), 0.0))
    out_ref[...] = jnp.full((1, 1), jnp.sum(term) * inv_b, jnp.float32)


def _tail(m1, xl, m2, xr, mem_mask, inv_b):
    b_sc, t = m1.shape
    body = functools.partial(_tail_body, b=b_sc, t=t, inv_b=inv_b)
    return pl.pallas_call(
        body,
        out_shape=jax.ShapeDtypeStruct((1, 1), jnp.float32),
    )(m1, xl, m2, xr, mem_mask[:b_sc])


@jax.jit
def kernel(inters, rels, labels, mem_mask, rels_label, gt_tracks,
           multilab_weights):
    b, t, c = inters.shape
    inv_b = 1.0 / b
    m1, xl, m2, xr = _sc_reduce(inters, rels, labels, mem_mask, rels_label,
                                gt_tracks, multilab_weights, _B_SC)
    tc_out = _tc_part(inters, rels, labels, mem_mask, rels_label, gt_tracks,
                      multilab_weights, _B_SC, inv_b)
    sc_out = _tail(m1, xl, m2, xr, mem_mask, inv_b)
    return (sc_out + tc_out).reshape((1,))


# speedup vs baseline: 4.6033x; 1.3394x over previous
import functools
import jax
import jax.numpy as jnp
from jax import lax
from jax.experimental import pallas as pl
from jax.experimental.pallas import tpu as pltpu
from jax.experimental.pallas import tpu_sc as plsc

_M = 0.2
_LYMBDA = 1.0
_NEG = float("-inf")
_L = 16
_NW = 32
_B_SC = 256


def _sig(x):
    return 1.0 / (1.0 + jnp.exp(-x))


def _sc_body(inters_h, rels_h, labels_h, mem_h, rl_h, gt_h, mw_h,
             m1_h, xl_h, m2_h, xr_h,
             xbuf, rbuf, mw_v, bias_v, rbias_v, lab_v, gt_v, mm_v, rlv_v,
             o_m1, o_xl, o_m2, o_xr, sem_x0, sem_x1,
             *, t, c, nr, nb):
    cid = lax.axis_index("c")
    sid = lax.axis_index("s")
    wid = sid * 2 + cid
    b0 = wid * nb

    pltpu.sync_copy(labels_h.at[pl.ds(b0, nb)], lab_v.at[pl.ds(0, nb)])
    pltpu.sync_copy(gt_h.at[pl.ds(b0, nb), :], gt_v.at[:, pl.ds(0, 2)])
    pltpu.sync_copy(mem_h.at[pl.ds(b0, nb), :], mm_v)
    pltpu.sync_copy(rl_h.at[pl.ds(b0, nb), :], rlv_v.at[:, pl.ds(0, t)])

    tio = lax.iota(jnp.int32, _L)
    ninf16 = jnp.full((_L,), _NEG, jnp.float32)
    ngroups = t // _L

    pltpu.async_copy(inters_h.at[b0, pl.ds(0, _L), :], xbuf.at[0], sem_x0)

    def b_body(i, carry):
        b = b0 + i
        pltpu.sync_copy(mw_h.at[b, :], mw_v)
        pltpu.sync_copy(rels_h.at[b], rbuf)
        lab_s = lab_v[pl.ds(i, _L)][0]
        gtrow = gt_v[i, pl.ds(0, _L)]
        g0s = gtrow[0]
        g1s = gtrow[1]
        t0 = rlv_v[i, pl.ds(g0s, _L)][0]
        t1 = rlv_v[i, pl.ds(g1s, _L)][0]
        t0c = jnp.minimum(t0, nr - 1)

        def bias_body(j, carry2):
            mwch = mw_v[pl.ds(j * _L, _L)]
            ci = lax.iota(jnp.int32, _L) + j * _L
            bias_v[pl.ds(j * _L, _L)] = jnp.where(
                (mwch > 0) & (ci != lab_s), 0.0, _NEG)
            return carry2

        lax.fori_loop(0, c // _L, bias_body, 0)

        for j in range(nr // _L):
            ci = lax.iota(jnp.int32, _L) + j * _L
            rbias_v[pl.ds(j * _L, _L)] = jnp.where(
                (ci != t0) & (ci != t1), 0.0, _NEG)

        for g in range(ngroups):
            buf = g % 2
            sem = sem_x0 if buf == 0 else sem_x1
            pltpu.make_async_copy(
                inters_h.at[b, pl.ds(g * _L, _L), :], xbuf.at[buf], sem
            ).wait()
            if g < ngroups - 1:
                nbuf = (g + 1) % 2
                nsem = sem_x0 if nbuf == 0 else sem_x1
                pltpu.async_copy(
                    inters_h.at[b, pl.ds((g + 1) * _L, _L), :],
                    xbuf.at[nbuf], nsem)
            else:
                @pl.when(i + 1 < nb)
                def _():
                    pltpu.async_copy(
                        inters_h.at[b + 1, pl.ds(0, _L), :], xbuf.at[0],
                        sem_x0)
            xb = xbuf.at[buf]
            mm_chunk = mm_v[i, pl.ds(g * _L, _L)]
            rl_chunk = rlv_v[i, pl.ds(g * _L, _L)]
            memok = mm_chunk > 0

            def cbody(j, accs):
                bch = bias_v[pl.ds(j * _L, _L)]
                out = []
                for tt in range(_L):
                    xv = xb[tt, pl.ds(j * _L, _L)]
                    out.append(jnp.maximum(accs[tt], xv + bch))
                return tuple(out)

            accs = lax.fori_loop(0, c // _L, cbody, (ninf16,) * _L)
            m1v = ninf16
            for tt in range(_L):
                m1v = jnp.where(tio == tt, jnp.max(accs[tt]), m1v)
            m1v = jnp.where(memok, m1v, _NEG)
            gxl = plsc.load_gather(xb, [tio, jnp.full((_L,), lab_s,
                                                      jnp.int32)])
            xlv = jnp.where(memok, gxl, _NEG)

            rtio = tio + g * _L

            raccs = [ninf16] * _L
            for j in range(nr // _L):
                rbch = rbias_v[pl.ds(j * _L, _L)]
                for tt in range(_L):
                    rv = rbuf[g * _L + tt, pl.ds(j * _L, _L)]
                    raccs[tt] = jnp.maximum(raccs[tt], rv + rbch)
            m2v = ninf16
            for tt in range(_L):
                m2v = jnp.where(tio == tt, jnp.max(raccs[tt]), m2v)
            rowok = memok & (rl_chunk != nr)
            m2v = jnp.where(rowok, m2v, _NEG)
            gxr = plsc.load_gather(rbuf, [rtio, jnp.full((_L,), t0c,
                                                         jnp.int32)])
            xrv = jnp.where(rowok & (t0 != nr), gxr, _NEG)

            o_m1[i, pl.ds(g * _L, _L)] = m1v
            o_xl[i, pl.ds(g * _L, _L)] = xlv
            o_m2[i, pl.ds(g * _L, _L)] = m2v
            o_xr[i, pl.ds(g * _L, _L)] = xrv
        return carry

    lax.fori_loop(0, nb, b_body, 0)

    pltpu.sync_copy(o_m1, m1_h.at[pl.ds(b0, nb), :])
    pltpu.sync_copy(o_xl, xl_h.at[pl.ds(b0, nb), :])
    pltpu.sync_copy(o_m2, m2_h.at[pl.ds(b0, nb), :])
    pltpu.sync_copy(o_xr, xr_h.at[pl.ds(b0, nb), :])


def _sc_reduce(inters, rels, labels, mem_mask, rels_label, gt_tracks,
               multilab_weights, b_sc):
    inters = inters[:b_sc]
    rels = rels[:b_sc]
    labels = labels[:b_sc]
    mem_mask = mem_mask[:b_sc]
    rels_label = rels_label[:b_sc]
    gt_tracks = gt_tracks[:b_sc]
    multilab_weights = multilab_weights[:b_sc]
    b, t, c = inters.shape
    nr = rels.shape[2]
    nb = b_sc // _NW
    mesh = plsc.VectorSubcoreMesh(core_axis_name="c", subcore_axis_name="s")
    body = functools.partial(_sc_body, t=t, c=c, nr=nr, nb=nb)
    f = pl.kernel(
        body,
        out_type=[jax.ShapeDtypeStruct((b_sc, t), jnp.float32)] * 4,
        mesh=mesh,
        compiler_params=pltpu.CompilerParams(
            use_tc_tiling_on_sc=False, needs_layout_passes=False),
        scratch_types=[
            pltpu.VMEM((2, _L, c), jnp.float32),
            pltpu.VMEM((t, nr), jnp.float32),
            pltpu.VMEM((c,), jnp.int32),
            pltpu.VMEM((c,), jnp.float32),
            pltpu.VMEM((nr,), jnp.float32),
            pltpu.VMEM((nb + _L,), jnp.int32),
            pltpu.VMEM((nb, _L), jnp.int32),
            pltpu.VMEM((nb, t), jnp.int32),
            pltpu.VMEM((nb, t + _L), jnp.int32),
            pltpu.VMEM((nb, t), jnp.float32),
            pltpu.VMEM((nb, t), jnp.float32),
            pltpu.VMEM((nb, t), jnp.float32),
            pltpu.VMEM((nb, t), jnp.float32),
            pltpu.SemaphoreType.DMA,
            pltpu.SemaphoreType.DMA,
        ],
    )
    return f(inters, rels, labels, mem_mask, rels_label, gt_tracks,
             multilab_weights)


def _tc_body(inters_ref, rels_ref, labels_ref, mem_ref, rl_ref, gt_ref,
             mw_ref, out_ref, *, bb, t, c, nr, inv_b):
    x = inters_ref[...]
    memi = mem_ref[...]
    memb = memi > 0
    mem3 = memi[:, :, None] > 0
    lab = labels_ref[...][:, 0]
    mw3 = mw_ref[...][:, None, :] > 0

    citer = lax.broadcasted_iota(jnp.int32, (bb, t, c), 2)
    tgt = citer == lab[:, None, None]
    negmask = mem3 & mw3 & (~tgt)
    m1 = jnp.max(jnp.where(negmask, x, _NEG), axis=2)
    xl = jnp.max(jnp.where(tgt & mem3, x, _NEG), axis=2)

    r = rels_ref[...]
    rl = rl_ref[...]
    rfi = memi * (rl != nr).astype(jnp.int32)
    rf3 = rfi[:, :, None] > 0
    g0 = gt_ref[...][:, 0]
    g1 = gt_ref[...][:, 1]
    titer = lax.broadcasted_iota(jnp.int32, (bb, t), 1)
    rel_t0 = jnp.sum(jnp.where(titer == g0[:, None], rl, 0), axis=1)
    rel_t1 = jnp.sum(jnp.where(titer == g1[:, None], rl, 0), axis=1)

    riter = lax.broadcasted_iota(jnp.int32, (bb, t, nr), 2)
    rneg = (rf3 & (riter != rel_t0[:, None, None])
            & (riter != rel_t1[:, None, None]))
    m2 = jnp.max(jnp.where(rneg, r, _NEG), axis=2)
    xr = jnp.max(jnp.where(rf3 & (riter == rel_t0[:, None, None]), r, _NEG),
                 axis=2)

    s_xl = _sig(xl)
    s_xr = _sig(xr)
    mv = (s_xl + s_xr) * memb.astype(jnp.float32)
    maxv = jnp.max(mv, axis=1)
    ismax = mv == maxv[:, None]
    first = jnp.min(jnp.where(ismax, titer, t), axis=1)
    sel = titer == first[:, None]
    pos = jnp.max(jnp.where(sel, s_xl, 0.0), axis=1)
    pos_r = jnp.max(jnp.where(sel, s_xr, 0.0), axis=1)

    term = (_LYMBDA * jnp.maximum(_M - pos[:, None] + _sig(m1), 0.0)
            + jnp.maximum(_M - pos_r[:, None] + _sig(m2), 0.0))
    partial = jnp.full((1, 1), jnp.sum(term) * inv_b, jnp.float32)

    @pl.when(pl.program_id(0) == 0)
    def _init():
        out_ref[...] = jnp.zeros((1, 1), jnp.float32)

    out_ref[...] += partial


def _tc_part(inters, rels, labels, mem_mask, rels_label, gt_tracks,
             multilab_weights, b_sc, inv_b):
    b, t, c = inters.shape
    nr = rels.shape[2]
    bb = 64
    k0 = b_sc // bb
    grid = ((b - b_sc) // bb,)
    body = functools.partial(_tc_body, bb=bb, t=t, c=c, nr=nr, inv_b=inv_b)
    return pl.pallas_call(
        body,
        grid=grid,
        in_specs=[
            pl.BlockSpec((bb, t, c), lambda i: (i + k0, 0, 0)),
            pl.BlockSpec((bb, t, nr), lambda i: (i + k0, 0, 0)),
            pl.BlockSpec((bb, 1), lambda i: (i + k0, 0)),
            pl.BlockSpec((bb, t), lambda i: (i + k0, 0)),
            pl.BlockSpec((bb, t), lambda i: (i + k0, 0)),
            pl.BlockSpec((bb, 2), lambda i: (i + k0, 0)),
            pl.BlockSpec((bb, c), lambda i: (i + k0, 0)),
        ],
        out_specs=pl.BlockSpec((1, 1), lambda i: (0, 0)),
        out_shape=jax.ShapeDtypeStruct((1, 1), jnp.float32),
    )(inters, rels, labels[:, None], mem_mask, rels_label, gt_tracks,
      multilab_weights)


def _tail_body(m1_ref, xl_ref, m2_ref, xr_ref, mem_ref, out_ref, *, b, t,
               inv_b):
    memb = mem_ref[...] > 0
    s_xl = _sig(xl_ref[...])
    s_xr = _sig(xr_ref[...])
    mv = (s_xl + s_xr) * memb.astype(jnp.float32)
    titer = lax.broadcasted_iota(jnp.int32, (b, t), 1)
    maxv = jnp.max(mv, axis=1)
    ismax = mv == maxv[:, None]
    first = jnp.min(jnp.where(ismax, titer, t), axis=1)
    sel = titer == first[:, None]
    pos = jnp.max(jnp.where(sel, s_xl, 0.0), axis=1)
    pos_r = jnp.max(jnp.where(sel, s_xr, 0.0), axis=1)
    term = (_LYMBDA * jnp.maximum(_M - pos[:, None] + _sig(m1_ref[...]), 0.0)
            + jnp.maximum(_M - pos_r[:, None] + _sig(m2_ref[...]), 0.0))
    out_ref[...] = jnp.full((1, 1), jnp.sum(term) * inv_b, jnp.float32)


def _tail(m1, xl, m2, xr, mem_mask, inv_b):
    b_sc, t = m1.shape
    body = functools.partial(_tail_body, b=b_sc, t=t, inv_b=inv_b)
    return pl.pallas_call(
        body,
        out_shape=jax.ShapeDtypeStruct((1, 1), jnp.float32),
    )(m1, xl, m2, xr, mem_mask[:b_sc])


@jax.jit
def kernel(inters, rels, labels, mem_mask, rels_label, gt_tracks,
           multilab_weights):
    b, t, c = inters.shape
    inv_b = 1.0 / b
    m1, xl, m2, xr = _sc_reduce(inters, rels, labels, mem_mask, rels_label,
                                gt_tracks, multilab_weights, _B_SC)
    tc_out = _tc_part(inters, rels, labels, mem_mask, rels_label, gt_tracks,
                      multilab_weights, _B_SC, inv_b)
    sc_out = _tail(m1, xl, m2, xr, mem_mask, inv_b)
    return (sc_out + tc_out).reshape((1,))


# speedup vs baseline: 6.5575x; 1.4245x over previous
import functools
import jax
import jax.numpy as jnp
from jax import lax
from jax.experimental import pallas as pl
from jax.experimental.pallas import tpu as pltpu
from jax.experimental.pallas import tpu_sc as plsc

_M = 0.2
_LYMBDA = 1.0
_NEG = float("-inf")
_L = 16
_NW = 32
_B_SC = 256


def _sig(x):
    return 1.0 / (1.0 + jnp.exp(-x))


def _sc_body(inters_h, rels_h, meta_h, mem_h, rl_h, mw_h,
             m1_h, xl_h, m2_h, xr_h,
             xbuf, rbuf, mw_v, bias_v, rbias_v, meta_v, mm_v, rlv_v,
             o_m1, o_xl, o_m2, o_xr, sem_x0, sem_x1,
             *, t, c, nr, nb):
    cid = lax.axis_index("c")
    sid = lax.axis_index("s")
    wid = sid * 2 + cid
    b0 = wid * nb

    pltpu.sync_copy(meta_h.at[pl.ds(b0, nb), :], meta_v)
    pltpu.sync_copy(mem_h.at[pl.ds(b0, nb), :], mm_v)
    pltpu.sync_copy(rl_h.at[pl.ds(b0, nb), :], rlv_v)

    tio = lax.iota(jnp.int32, _L)
    ninf16 = jnp.full((_L,), _NEG, jnp.float32)
    ngroups = t // _L

    pltpu.async_copy(inters_h.at[b0, pl.ds(0, _L), :], xbuf.at[0], sem_x0)

    def b_body(i, carry):
        b = b0 + i
        pltpu.sync_copy(mw_h.at[b, :], mw_v)
        pltpu.sync_copy(rels_h.at[b], rbuf)
        mrow = meta_v[i, pl.ds(0, _L)]
        lab_s = mrow[0]
        t0 = mrow[1]
        t1 = mrow[2]
        t0c = jnp.minimum(t0, nr - 1)

        def bias_body(j, carry2):
            mwch = mw_v[pl.ds(j * _L, _L)]
            ci = lax.iota(jnp.int32, _L) + j * _L
            bias_v[pl.ds(j * _L, _L)] = jnp.where(
                (mwch > 0) & (ci != lab_s), 0.0, _NEG)
            return carry2

        lax.fori_loop(0, c // _L, bias_body, 0)

        for j in range(nr // _L):
            ci = lax.iota(jnp.int32, _L) + j * _L
            rbias_v[pl.ds(j * _L, _L)] = jnp.where(
                (ci != t0) & (ci != t1), 0.0, _NEG)

        for g in range(ngroups):
            buf = g % 2
            sem = sem_x0 if buf == 0 else sem_x1
            pltpu.make_async_copy(
                inters_h.at[b, pl.ds(g * _L, _L), :], xbuf.at[buf], sem
            ).wait()
            if g < ngroups - 1:
                nbuf = (g + 1) % 2
                nsem = sem_x0 if nbuf == 0 else sem_x1
                pltpu.async_copy(
                    inters_h.at[b, pl.ds((g + 1) * _L, _L), :],
                    xbuf.at[nbuf], nsem)
            else:
                @pl.when(i + 1 < nb)
                def _():
                    pltpu.async_copy(
                        inters_h.at[b + 1, pl.ds(0, _L), :], xbuf.at[0],
                        sem_x0)
            xb = xbuf.at[buf]
            mm_chunk = mm_v[i, pl.ds(g * _L, _L)]
            rl_chunk = rlv_v[i, pl.ds(g * _L, _L)]
            memok = mm_chunk > 0

            def cbody(j, accs):
                bch = bias_v[pl.ds(j * _L, _L)]
                out = []
                for tt in range(_L):
                    xv = xb[tt, pl.ds(j * _L, _L)]
                    out.append(jnp.maximum(accs[tt], xv + bch))
                return tuple(out)

            accs = lax.fori_loop(0, c // _L, cbody, (ninf16,) * _L)
            labj = (lab_s // _L) * _L
            labbias = jnp.where(tio == lab_s - labj, 0.0, _NEG)
            m1v = ninf16
            gxl = ninf16
            for tt in range(_L):
                m1v = jnp.where(tio == tt, jnp.max(accs[tt]), m1v)
                xlch = xb[tt, pl.ds(labj, _L)]
                gxl = jnp.where(tio == tt, jnp.max(xlch + labbias), gxl)
            m1v = jnp.where(memok, m1v, _NEG)
            xlv = jnp.where(memok, gxl, _NEG)

            raccs = [ninf16] * _L
            for j in range(nr // _L):
                rbch = rbias_v[pl.ds(j * _L, _L)]
                for tt in range(_L):
                    rv = rbuf[g * _L + tt, pl.ds(j * _L, _L)]
                    raccs[tt] = jnp.maximum(raccs[tt], rv + rbch)
            t0j = (t0c // _L) * _L
            t0bias = jnp.where(tio == t0c - t0j, 0.0, _NEG)
            m2v = ninf16
            gxr = ninf16
            for tt in range(_L):
                m2v = jnp.where(tio == tt, jnp.max(raccs[tt]), m2v)
                xrch = rbuf[g * _L + tt, pl.ds(t0j, _L)]
                gxr = jnp.where(tio == tt, jnp.max(xrch + t0bias), gxr)
            rowok = memok & (rl_chunk != nr)
            m2v = jnp.where(rowok, m2v, _NEG)
            xrv = jnp.where(rowok & (t0 != nr), gxr, _NEG)

            o_m1[i, pl.ds(g * _L, _L)] = m1v
            o_xl[i, pl.ds(g * _L, _L)] = xlv
            o_m2[i, pl.ds(g * _L, _L)] = m2v
            o_xr[i, pl.ds(g * _L, _L)] = xrv
        return carry

    lax.fori_loop(0, nb, b_body, 0)

    pltpu.sync_copy(o_m1, m1_h.at[pl.ds(b0, nb), :])
    pltpu.sync_copy(o_xl, xl_h.at[pl.ds(b0, nb), :])
    pltpu.sync_copy(o_m2, m2_h.at[pl.ds(b0, nb), :])
    pltpu.sync_copy(o_xr, xr_h.at[pl.ds(b0, nb), :])


def _sc_reduce(inters, rels, meta, mem128, rl128, multilab_weights, b_sc):
    b, t, c = inters.shape
    nr = rels.shape[2]
    nb = b_sc // _NW
    mesh = plsc.VectorSubcoreMesh(core_axis_name="c", subcore_axis_name="s")
    body = functools.partial(_sc_body, t=t, c=c, nr=nr, nb=nb)
    f = pl.kernel(
        body,
        out_type=[jax.ShapeDtypeStruct((b_sc, 128), jnp.float32)] * 4,
        mesh=mesh,
        compiler_params=pltpu.CompilerParams(needs_layout_passes=False),
        scratch_types=[
            pltpu.VMEM((2, _L, c), jnp.float32),
            pltpu.VMEM((t, nr), jnp.float32),
            pltpu.VMEM((c,), jnp.int32),
            pltpu.VMEM((c,), jnp.float32),
            pltpu.VMEM((nr,), jnp.float32),
            pltpu.VMEM((nb, 128), jnp.int32),
            pltpu.VMEM((nb, 128), jnp.int32),
            pltpu.VMEM((nb, 128), jnp.int32),
            pltpu.VMEM((nb, 128), jnp.float32),
            pltpu.VMEM((nb, 128), jnp.float32),
            pltpu.VMEM((nb, 128), jnp.float32),
            pltpu.VMEM((nb, 128), jnp.float32),
            pltpu.SemaphoreType.DMA,
            pltpu.SemaphoreType.DMA,
        ],
    )
    return f(inters, rels, meta, mem128, rl128, multilab_weights)


def _tc_body(inters_ref, rels_ref, labels_ref, mem_ref, rl_ref, gt_ref,
             mw_ref, out_ref, *, bb, t, c, nr, inv_b):
    x = inters_ref[...]
    memi = mem_ref[...]
    memb = memi > 0
    mem3 = memi[:, :, None] > 0
    lab = labels_ref[...][:, 0]
    mw3 = mw_ref[...][:, None, :] > 0

    citer = lax.broadcasted_iota(jnp.int32, (bb, t, c), 2)
    tgt = citer == lab[:, None, None]
    negmask = mem3 & mw3 & (~tgt)
    m1 = jnp.max(jnp.where(negmask, x, _NEG), axis=2)
    xl = jnp.max(jnp.where(tgt & mem3, x, _NEG), axis=2)

    r = rels_ref[...]
    rl = rl_ref[...]
    rfi = memi * (rl != nr).astype(jnp.int32)
    rf3 = rfi[:, :, None] > 0
    g0 = gt_ref[...][:, 0]
    g1 = gt_ref[...][:, 1]
    titer = lax.broadcasted_iota(jnp.int32, (bb, t), 1)
    rel_t0 = jnp.sum(jnp.where(titer == g0[:, None], rl, 0), axis=1)
    rel_t1 = jnp.sum(jnp.where(titer == g1[:, None], rl, 0), axis=1)

    riter = lax.broadcasted_iota(jnp.int32, (bb, t, nr), 2)
    rneg = (rf3 & (riter != rel_t0[:, None, None])
            & (riter != rel_t1[:, None, None]))
    m2 = jnp.max(jnp.where(rneg, r, _NEG), axis=2)
    xr = jnp.max(jnp.where(rf3 & (riter == rel_t0[:, None, None]), r, _NEG),
                 axis=2)

    s_xl = _sig(xl)
    s_xr = _sig(xr)
    mv = (s_xl + s_xr) * memb.astype(jnp.float32)
    maxv = jnp.max(mv, axis=1)
    ismax = mv == maxv[:, None]
    first = jnp.min(jnp.where(ismax, titer, t), axis=1)
    sel = titer == first[:, None]
    pos = jnp.max(jnp.where(sel, s_xl, 0.0), axis=1)
    pos_r = jnp.max(jnp.where(sel, s_xr, 0.0), axis=1)

    term = (_LYMBDA * jnp.maximum(_M - pos[:, None] + _sig(m1), 0.0)
            + jnp.maximum(_M - pos_r[:, None] + _sig(m2), 0.0))
    partial = jnp.full((1, 1), jnp.sum(term) * inv_b, jnp.float32)

    @pl.when(pl.program_id(0) == 0)
    def _init():
        out_ref[...] = jnp.zeros((1, 1), jnp.float32)

    out_ref[...] += partial


def _tc_part(inters, rels, labels, mem_mask, rels_label, gt_tracks,
             multilab_weights, b_sc, inv_b):
    b, t, c = inters.shape
    nr = rels.shape[2]
    bb = 64
    k0 = b_sc // bb
    grid = ((b - b_sc) // bb,)
    body = functools.partial(_tc_body, bb=bb, t=t, c=c, nr=nr, inv_b=inv_b)
    return pl.pallas_call(
        body,
        grid=grid,
        in_specs=[
            pl.BlockSpec((bb, t, c), lambda i: (i + k0, 0, 0)),
            pl.BlockSpec((bb, t, nr), lambda i: (i + k0, 0, 0)),
            pl.BlockSpec((bb, 1), lambda i: (i + k0, 0)),
            pl.BlockSpec((bb, t), lambda i: (i + k0, 0)),
            pl.BlockSpec((bb, t), lambda i: (i + k0, 0)),
            pl.BlockSpec((bb, 2), lambda i: (i + k0, 0)),
            pl.BlockSpec((bb, c), lambda i: (i + k0, 0)),
        ],
        out_specs=pl.BlockSpec((1, 1), lambda i: (0, 0)),
        out_shape=jax.ShapeDtypeStruct((1, 1), jnp.float32),
    )(inters, rels, labels[:, None], mem_mask, rels_label, gt_tracks,
      multilab_weights)


def _tail_body(m1_ref, xl_ref, m2_ref, xr_ref, mem_ref, out_ref, *, b, t,
               inv_b):
    memb = mem_ref[...] > 0
    s_xl = _sig(xl_ref[...])
    s_xr = _sig(xr_ref[...])
    mv = (s_xl + s_xr) * memb.astype(jnp.float32)
    titer = lax.broadcasted_iota(jnp.int32, (b, t), 1)
    maxv = jnp.max(mv, axis=1)
    ismax = mv == maxv[:, None]
    first = jnp.min(jnp.where(ismax, titer, t), axis=1)
    sel = titer == first[:, None]
    pos = jnp.max(jnp.where(sel, s_xl, 0.0), axis=1)
    pos_r = jnp.max(jnp.where(sel, s_xr, 0.0), axis=1)
    term = (_LYMBDA * jnp.maximum(_M - pos[:, None] + _sig(m1_ref[...]), 0.0)
            + jnp.maximum(_M - pos_r[:, None] + _sig(m2_ref[...]), 0.0))
    out_ref[...] = jnp.full((1, 1), jnp.sum(term) * inv_b, jnp.float32)


def _tail(m1, xl, m2, xr, mem_mask, inv_b):
    b_sc, t = m1.shape
    body = functools.partial(_tail_body, b=b_sc, t=t, inv_b=inv_b)
    return pl.pallas_call(
        body,
        out_shape=jax.ShapeDtypeStruct((1, 1), jnp.float32),
    )(m1, xl, m2, xr, mem_mask[:b_sc])


@jax.jit
def kernel(inters, rels, labels, mem_mask, rels_label, gt_tracks,
           multilab_weights):
    b, t, c = inters.shape
    inv_b = 1.0 / b
    t01 = jnp.take_along_axis(rels_label[:_B_SC], gt_tracks[:_B_SC], axis=1)
    meta = jnp.concatenate(
        [labels[:_B_SC, None], t01,
         jnp.zeros((_B_SC, 125), jnp.int32)], axis=1)
    pad = ((0, 0), (0, 128 - t))
    mem128 = jnp.pad(mem_mask[:_B_SC], pad)
    rl128 = jnp.pad(rels_label[:_B_SC], pad)
    m1, xl, m2, xr = _sc_reduce(inters, rels, meta, mem128, rl128,
                                multilab_weights, _B_SC)
    tc_out = _tc_part(inters, rels, labels, mem_mask, rels_label, gt_tracks,
                      multilab_weights, _B_SC, inv_b)
    sc_out = _tail(m1[:, :t], xl[:, :t], m2[:, :t], xr[:, :t], mem_mask,
                   inv_b)
    return (sc_out + tc_out).reshape((1,))
